# Initial kernel scaffold; baseline (speedup 1.0000x reference)
import jax
import jax.numpy as jnp
from jax.experimental import pallas as pl


def kernel(node_feat, edge_index, edge_attr, batch_index, shared_net, layers):
    raise NotImplementedError("write your pallas kernel here")


# speedup vs baseline: 2.4314x; 2.4314x over previous
import functools

import jax
import jax.numpy as jnp
import numpy as np
from jax import lax
from jax.experimental import pallas as pl
from jax.experimental.pallas import tpu as pltpu
from jax.experimental.pallas import tpu_sc as plsc

N = 10000
E = 160000
D = 16
EPS = 1e-5
SLOPE = 0.01

EP = E // 8
BB = 1000
GRID_E = EP // BB
NW = 32
WIN = 128
NWIN = E // WIN
WPW = -(-NWIN // NW)
N_PAD = 10240
NR = N_PAD // 16
NP = N_PAD // 8


def _lrelu(x):
    return jnp.where(x >= 0, x, SLOPE * x)


def _fold(v):
    s = v[:, 0:D]
    for j in range(1, 8):
        s = s + v[:, D * j:D * j + D]
    return jnp.concatenate([s] * 8, axis=1)


def _finalize(st_ref, g_ref, be_ref):
    m = _fold(st_ref[0:1, :]) * (1.0 / E)
    q = _fold(st_ref[1:2, :]) * (1.0 / E)
    inv = lax.rsqrt(q - m * m + EPS)
    return m, inv * g_ref[...], be_ref[...]


def _acc_stats(acc, a, slot, is_first, is_last, st_ref):
    @pl.when(is_first)
    def _():
        acc[2 * slot:2 * slot + 2, :] = jnp.zeros((2, 128), jnp.float32)

    acc[2 * slot:2 * slot + 1, :] += jnp.sum(a, axis=0, keepdims=True)
    acc[2 * slot + 1:2 * slot + 2, :] += jnp.sum(a * a, axis=0, keepdims=True)

    @pl.when(is_last)
    def _():
        st_ref[...] = acc[2 * slot:2 * slot + 2, :]


def _eblk(i):
    return (i, 0)


def _const(*_):
    return (0, 0)


_EB = pl.BlockSpec((BB, 128), _eblk)
_CONST2 = pl.BlockSpec((2, 128), _const)
_CONST1 = pl.BlockSpec((1, 128), _const)
_WBD = pl.BlockSpec((128, 128), _const)
_E16 = jax.ShapeDtypeStruct((EP, 128), jnp.float32)
_ST = jax.ShapeDtypeStruct((2, 128), jnp.float32)
_TC_PARAMS = pltpu.CompilerParams(dimension_semantics=("arbitrary",))


def _s1_body(ea, xs, xd, w1a, w1b, w1c, a1, st, acc):
    i = pl.program_id(0)
    a = ea[...] @ w1a[...] + xs[...] @ w1b[...] + xd[...] @ w1c[...]
    a1[...] = a
    _acc_stats(acc, a, 0, i == 0, i == GRID_E - 1, st)


def _s1(ea, xs, xd, w1a, w1b, w1c):
    return pl.pallas_call(
        _s1_body,
        grid=(GRID_E,),
        in_specs=[pl.BlockSpec((BB, 32), _eblk), _EB, _EB,
                  pl.BlockSpec((32, 128), _const), _WBD, _WBD],
        out_specs=[_EB, _CONST2],
        out_shape=[_E16, _ST],
        scratch_shapes=[pltpu.VMEM((2, 128), jnp.float32)],
        compiler_params=_TC_PARAMS,
    )(ea, xs, xd, w1a, w1b, w1c)


def _s2_body(a1, st1, g1, be1, w2, a2, h1o, st, acc):
    i = pl.program_id(0)
    m, s, b = _finalize(st1, g1, be1)
    h1 = _lrelu((a1[...] - m) * s + b)
    h1o[...] = h1
    a = h1 @ w2[...]
    a2[...] = a
    _acc_stats(acc, a, 0, i == 0, i == GRID_E - 1, st)


def _s2(a1, st1, g1, be1, w2):
    return pl.pallas_call(
        _s2_body,
        grid=(GRID_E,),
        in_specs=[_EB, _CONST2, _CONST1, _CONST1, _WBD],
        out_specs=[_EB, _EB, _CONST2],
        out_shape=[_E16, _E16, _ST],
        scratch_shapes=[pltpu.VMEM((2, 128), jnp.float32)],
        compiler_params=_TC_PARAMS,
    )(a1, st1, g1, be1, w2)


def _s3_body(a2, h1, st2, g2, be2, w3, a3, h2o, st, acc):
    i = pl.program_id(0)
    m, s, b = _finalize(st2, g2, be2)
    h2 = _lrelu((a2[...] - m) * s + b) + h1[...]
    h2o[...] = h2
    a = h2 @ w3[...]
    a3[...] = a
    _acc_stats(acc, a, 0, i == 0, i == GRID_E - 1, st)


def _s3(a2, h1, st2, g2, be2, w3):
    return pl.pallas_call(
        _s3_body,
        grid=(GRID_E,),
        in_specs=[_EB, _EB, _CONST2, _CONST1, _CONST1, _WBD],
        out_specs=[_EB, _EB, _CONST2],
        out_shape=[_E16, _E16, _ST],
        scratch_shapes=[pltpu.VMEM((2, 128), jnp.float32)],
        compiler_params=_TC_PARAMS,
    )(a2, h1, st2, g2, be2, w3)


def _s4_body(a3, h2, st3, g3, be3, u1, u2, efo, a4, a5, st4, st5, acc):
    i = pl.program_id(0)
    m, s, b = _finalize(st3, g3, be3)
    ef = _lrelu((a3[...] - m) * s + b) + h2[...]
    efo[...] = ef
    x4 = ef @ u1[...]
    a4[...] = x4
    _acc_stats(acc, x4, 0, i == 0, i == GRID_E - 1, st4)
    x5 = ef @ u2[...]
    a5[...] = x5
    _acc_stats(acc, x5, 1, i == 0, i == GRID_E - 1, st5)


def _s4(a3, h2, st3, g3, be3, u1, u2):
    return pl.pallas_call(
        _s4_body,
        grid=(GRID_E,),
        in_specs=[_EB, _EB, _CONST2, _CONST1, _CONST1, _WBD, _WBD],
        out_specs=[_EB, _EB, _EB, _CONST2, _CONST2],
        out_shape=[_E16, _E16, _E16, _ST, _ST],
        scratch_shapes=[pltpu.VMEM((4, 128), jnp.float32)],
        compiler_params=_TC_PARAMS,
    )(a3, h2, st3, g3, be3, u1, u2)


def _s5_body(a4, a5, ef, st4, g4, be4, st5, g5, be5, wos, bos, msg1, h5o):
    m4, s4, b4 = _finalize(st4, g4, be4)
    efv = ef[...]
    h4 = _lrelu((a4[...] - m4) * s4 + b4) + efv
    msg1[...] = h4 @ wos[...] + bos[...]
    m5, s5, b5 = _finalize(st5, g5, be5)
    h5o[...] = _lrelu((a5[...] - m5) * s5 + b5) + efv


def _s5(a4, a5, ef, st4, g4, be4, st5, g5, be5, wos, bos):
    return pl.pallas_call(
        _s5_body,
        grid=(GRID_E,),
        in_specs=[_EB, _EB, _EB, _CONST2, _CONST1, _CONST1,
                  _CONST2, _CONST1, _CONST1, _WBD, _CONST1],
        out_specs=[_EB, _EB],
        out_shape=[_E16, _E16],
        compiler_params=_TC_PARAMS,
    )(a4, a5, ef, st4, g4, be4, st5, g5, be5, wos, bos)


def _nstats(t):
    rows = lax.broadcasted_iota(jnp.int32, (NP, 128), 0)
    tm = jnp.where(rows < N // 8, t, 0.0)
    m = _fold(jnp.sum(tm, axis=0, keepdims=True)) * (1.0 / N)
    q = _fold(jnp.sum(tm * tm, axis=0, keepdims=True)) * (1.0 / N)
    return m, q


def _na_body(p0, p1, g, be, rw2, out1o, root2o):
    aggr = p0[...] + p1[...]
    m, q = _nstats(aggr)
    inv = lax.rsqrt(q - m * m + EPS)
    out1 = _lrelu((aggr - m) * (inv * g[...]) + be[...]) + 1.0
    out1o[...] = out1
    root2o[...] = out1 @ rw2[...]


def _na(p0, p1, g, be, rw2):
    nb = pl.BlockSpec((NP, 128), _const)
    return pl.pallas_call(
        _na_body,
        grid=(1,),
        in_specs=[nb, nb, _CONST1, _CONST1, _WBD],
        out_specs=[nb, nb],
        out_shape=[jax.ShapeDtypeStruct((NP, 128), jnp.float32),
                   jax.ShapeDtypeStruct((NP, 128), jnp.float32)],
    )(p0, p1, g, be, rw2)


def _s10_body(h5, xs2, wo2, bo2, msg2):
    h5v = h5[...]
    xsv = xs2[...]
    for j in range(8):
        h5j = h5v[:, D * j:D * j + D]
        xsj = xsv[:, D * j:D * j + D]
        wblk = h5j @ wo2[...] + bo2[...]
        acc = xsj[:, 0:1] * wblk[:, 0:D]
        for i in range(1, D):
            acc = acc + xsj[:, i:i + 1] * wblk[:, D * i:D * i + D]
        msg2[:, D * j:D * j + D] = acc


def _s10(h5, xs2, wo2, bo2):
    return pl.pallas_call(
        _s10_body,
        grid=(GRID_E,),
        in_specs=[_EB, _EB, pl.BlockSpec((D, 16 * D), _const),
                  pl.BlockSpec((1, 16 * D), _const)],
        out_specs=[_EB],
        out_shape=[_E16],
        compiler_params=_TC_PARAMS,
    )(h5, xs2, wo2, bo2)


def _nb_body(p0, p1, root2, out1, g, be, out2o):
    t = p0[...] + p1[...] + root2[...]
    m, q = _nstats(t)
    inv = lax.rsqrt(q - m * m + EPS)
    out2o[...] = _lrelu((t - m) * (inv * g[...]) + be[...]) + out1[...]


def _nb(p0, p1, root2, out1, g, be):
    nb = pl.BlockSpec((NP, 128), _const)
    return pl.pallas_call(
        _nb_body,
        grid=(1,),
        in_specs=[nb, nb, nb, nb, _CONST1, _CONST1],
        out_specs=nb,
        out_shape=jax.ShapeDtypeStruct((NP, 128), jnp.float32),
    )(p0, p1, root2, out1, g, be)


_SC_MESH = plsc.VectorSubcoreMesh(core_axis_name="core", subcore_axis_name="subcore")
_SC_PARAMS = pltpu.CompilerParams(use_tc_tiling_on_sc=False)


def _gather2(table, src1, dst1):
    @functools.partial(
        pl.kernel,
        out_type=[jax.ShapeDtypeStruct((E, D), jnp.float32),
                  jax.ShapeDtypeStruct((E, D), jnp.float32)],
        mesh=_SC_MESH,
        compiler_params=_SC_PARAMS,
    )
    def k(tab_hbm, src_hbm, dst_hbm, xs_hbm, xd_hbm):
        def body(is_v, id_v, xs_v, xd_v):
            pltpu.sync_copy(tab_hbm.at[is_v.at[0]], xs_v)
            pltpu.sync_copy(tab_hbm.at[id_v.at[0]], xd_v)

        pltpu.emit_pipeline(
            body,
            grid=(NWIN,),
            in_specs=[pl.BlockSpec((1, WIN), lambda i: (0, i)),
                      pl.BlockSpec((1, WIN), lambda i: (0, i))],
            out_specs=[pl.BlockSpec((WIN, D), lambda i: (i, 0)),
                       pl.BlockSpec((WIN, D), lambda i: (i, 0))],
            core_axis_name=("core", "subcore"),
            dimension_semantics=(pltpu.PARALLEL,),
        )(src_hbm, dst_hbm, xs_hbm, xd_hbm)

    return k(table, src1, dst1)


def _gather1(table, idx):
    @functools.partial(
        pl.kernel,
        out_type=jax.ShapeDtypeStruct((E, D), jnp.float32),
        mesh=_SC_MESH,
        compiler_params=_SC_PARAMS,
    )
    def k(tab_hbm, idx_hbm, out_hbm):
        def body(i_v, o_v):
            pltpu.sync_copy(tab_hbm.at[i_v.at[0]], o_v)

        pltpu.emit_pipeline(
            body,
            grid=(NWIN,),
            in_specs=[pl.BlockSpec((1, WIN), lambda i: (0, i))],
            out_specs=[pl.BlockSpec((WIN, D), lambda i: (i, 0))],
            core_axis_name=("core", "subcore"),
            dimension_semantics=(pltpu.PARALLEL,),
        )(idx_hbm, out_hbm)

    return k(table, idx)


def _scatter_add(msg, dst3, zeros):
    @functools.partial(
        pl.kernel,
        out_type=[jax.ShapeDtypeStruct((N_PAD, D), jnp.float32),
                  jax.ShapeDtypeStruct((N_PAD, D), jnp.float32)],
        mesh=_SC_MESH,
        scratch_types=[pltpu.VMEM((1, WIN), jnp.int32),
                       pltpu.VMEM((WIN, D), jnp.float32),
                       pltpu.VMEM((NR, D), jnp.float32),
                       pltpu.VMEM_SHARED((N_PAD, D), jnp.float32)],
        compiler_params=_SC_PARAMS,
    )
    def k(msg_hbm, dst_hbm, z_hbm, p0_hbm, p1_hbm, idx_v, msg_v, bounce_v, acc_sh):
        c = lax.axis_index("core")
        s = lax.axis_index("subcore")
        w = c * 16 + s
        row0 = pl.multiple_of(s * NR, 8)
        pltpu.sync_copy(z_hbm.at[pl.ds(row0, NR)], acc_sh.at[pl.ds(row0, NR)])
        plsc.subcore_barrier()

        @pl.loop(0, WPW)
        def _(j):
            win = w + j * NW

            @pl.when(win < NWIN)
            def _():
                e0 = pl.multiple_of(win * WIN, WIN)
                pltpu.sync_copy(dst_hbm.at[win], idx_v)
                pltpu.sync_copy(msg_hbm.at[pl.ds(e0, WIN)], msg_v)
                pltpu.sync_copy(msg_v, acc_sh.at[idx_v.at[0]], add=True)

        plsc.subcore_barrier()
        pltpu.sync_copy(acc_sh.at[pl.ds(row0, NR)], bounce_v)

        @pl.when(c == 0)
        def _():
            pltpu.sync_copy(bounce_v, p0_hbm.at[pl.ds(row0, NR)])

        @pl.when(c == 1)
        def _():
            pltpu.sync_copy(bounce_v, p1_hbm.at[pl.ds(row0, NR)])

    return k(msg, dst3, zeros)


def _bd8(w):
    return jnp.kron(jnp.eye(8, dtype=jnp.float32), w)


def _tile8(v):
    return jnp.tile(v.reshape(1, D), (1, 8))


def kernel(node_feat, edge_index, edge_attr, batch_index, shared_net, layers):
    del batch_index
    (w1, _, g1, be1), (w2, _, g2, be2), (w3, _, g3, be3) = shared_net
    w1a, w1b, w1c = w1[:4], w1[4:4 + D], w1[4 + D:4 + 2 * D]
    l1, l2 = layers
    (u1, _, ug1, ube1), = l1["en_hidden"]
    (u2, _, ug2, ube2), = l2["en_hidden"]
    wo1, bo1 = l1["en_out"]
    wos = wo1.reshape(D, D, D).sum(axis=1)
    bos = bo1.reshape(D, D).sum(axis=0)
    wo2, bo2 = l2["en_out"]
    bo2r = bo2.reshape(1, D * D)

    g1, be1, g2, be2, g3, be3 = map(_tile8, (g1, be1, g2, be2, g3, be3))
    ug1, ube1, ug2, ube2 = map(_tile8, (ug1, ube1, ug2, ube2))
    n1g, n1b = _tile8(l1["bn_g"]), _tile8(l1["bn_b"])
    n2g, n2b = _tile8(l2["bn_g"]), _tile8(l2["bn_b"])

    src1 = edge_index[0].reshape(1, E)
    dst1 = edge_index[1].reshape(1, E)
    dst3 = edge_index[1].reshape(NWIN, 1, WIN)
    zeros = jnp.zeros((N_PAD, D), jnp.float32)
    ea_p = edge_attr.reshape(EP, 32)

    pk = lambda x: x.reshape(EP, 128)
    pkn = lambda x: x.reshape(NP, 128)

    xs, xd = _gather2(node_feat, src1, dst1)
    a1, st1 = _s1(ea_p, pk(xs), pk(xd), _bd8(w1a), _bd8(w1b), _bd8(w1c))
    a2, h1, st2 = _s2(a1, st1, g1, be1, _bd8(w2))
    a3, h2, st3 = _s3(a2, h1, st2, g2, be2, _bd8(w3))
    ef, a4, a5, st4, st5 = _s4(a3, h2, st3, g3, be3, _bd8(u1), _bd8(u2))
    msg1, h5 = _s5(a4, a5, ef, st4, ug1, ube1, st5, ug2, ube2,
                   _bd8(wos), _tile8(bos))
    p0, p1 = _scatter_add(msg1.reshape(E, D), dst3, zeros)
    out1, root2 = _na(pkn(p0), pkn(p1), n1g, n1b, _bd8(l2["root_W"]))
    xs2 = _gather1(out1.reshape(N_PAD, D), src1)
    msg2, = _s10(h5, pk(xs2), wo2, bo2r)
    q0, q1 = _scatter_add(msg2.reshape(E, D), dst3, zeros)
    out2 = _nb(pkn(q0), pkn(q1), root2, out1, n2g, n2b)
    return out2.reshape(N_PAD, D)[:N]


# speedup vs baseline: 2.5602x; 1.0530x over previous
import functools

import jax
import jax.numpy as jnp
import numpy as np
from jax import lax
from jax.experimental import pallas as pl
from jax.experimental.pallas import tpu as pltpu
from jax.experimental.pallas import tpu_sc as plsc

N = 10000
E = 160000
D = 16
EPS = 1e-5
SLOPE = 0.01

EP = E // 8
BB = 1000
GRID_E = EP // BB
NW = 32
WIN = 128
NWIN = E // WIN
WPW = 40
NWIN_PAD = NW * WPW
E_PAD = NWIN_PAD * WIN
CH = 20
N_PAD = 10240
NR = N_PAD // 16
NP = N_PAD // 8


def _lrelu(x):
    return jnp.where(x >= 0, x, SLOPE * x)


def _fold(v):
    s = v[:, 0:D]
    for j in range(1, 8):
        s = s + v[:, D * j:D * j + D]
    return jnp.concatenate([s] * 8, axis=1)


def _finalize(st_ref, g_ref, be_ref):
    m = _fold(st_ref[0:1, :]) * (1.0 / E)
    q = _fold(st_ref[1:2, :]) * (1.0 / E)
    inv = lax.rsqrt(q - m * m + EPS)
    return m, inv * g_ref[...], be_ref[...]


def _acc_stats(acc, a, slot, is_first, is_last, st_ref):
    @pl.when(is_first)
    def _():
        acc[2 * slot:2 * slot + 2, :] = jnp.zeros((2, 128), jnp.float32)

    acc[2 * slot:2 * slot + 1, :] += jnp.sum(a, axis=0, keepdims=True)
    acc[2 * slot + 1:2 * slot + 2, :] += jnp.sum(a * a, axis=0, keepdims=True)

    @pl.when(is_last)
    def _():
        st_ref[...] = acc[2 * slot:2 * slot + 2, :]


def _eblk(i):
    return (i, 0)


def _const(*_):
    return (0, 0)


_EB = pl.BlockSpec((BB, 128), _eblk)
_CONST2 = pl.BlockSpec((2, 128), _const)
_CONST1 = pl.BlockSpec((1, 128), _const)
_WBD = pl.BlockSpec((128, 128), _const)
_E16 = jax.ShapeDtypeStruct((EP, 128), jnp.float32)
_ST = jax.ShapeDtypeStruct((2, 128), jnp.float32)
_TC_PARAMS = pltpu.CompilerParams(dimension_semantics=("arbitrary",))


def _s1_body(ea, xs, xd, w1a, w1b, w1c, a1, st, acc):
    i = pl.program_id(0)
    a = ea[...] @ w1a[...] + xs[...] @ w1b[...] + xd[...] @ w1c[...]
    a1[...] = a
    _acc_stats(acc, a, 0, i == 0, i == GRID_E - 1, st)


def _s1(ea, xs, xd, w1a, w1b, w1c):
    return pl.pallas_call(
        _s1_body,
        grid=(GRID_E,),
        in_specs=[pl.BlockSpec((BB, 32), _eblk), _EB, _EB,
                  pl.BlockSpec((32, 128), _const), _WBD, _WBD],
        out_specs=[_EB, _CONST2],
        out_shape=[_E16, _ST],
        scratch_shapes=[pltpu.VMEM((2, 128), jnp.float32)],
        compiler_params=_TC_PARAMS,
    )(ea, xs, xd, w1a, w1b, w1c)


def _s2_body(a1, st1, g1, be1, w2, a2, h1o, st, acc):
    i = pl.program_id(0)
    m, s, b = _finalize(st1, g1, be1)
    h1 = _lrelu((a1[...] - m) * s + b)
    h1o[...] = h1
    a = h1 @ w2[...]
    a2[...] = a
    _acc_stats(acc, a, 0, i == 0, i == GRID_E - 1, st)


def _s2(a1, st1, g1, be1, w2):
    return pl.pallas_call(
        _s2_body,
        grid=(GRID_E,),
        in_specs=[_EB, _CONST2, _CONST1, _CONST1, _WBD],
        out_specs=[_EB, _EB, _CONST2],
        out_shape=[_E16, _E16, _ST],
        scratch_shapes=[pltpu.VMEM((2, 128), jnp.float32)],
        compiler_params=_TC_PARAMS,
    )(a1, st1, g1, be1, w2)


def _s3_body(a2, h1, st2, g2, be2, w3, a3, h2o, st, acc):
    i = pl.program_id(0)
    m, s, b = _finalize(st2, g2, be2)
    h2 = _lrelu((a2[...] - m) * s + b) + h1[...]
    h2o[...] = h2
    a = h2 @ w3[...]
    a3[...] = a
    _acc_stats(acc, a, 0, i == 0, i == GRID_E - 1, st)


def _s3(a2, h1, st2, g2, be2, w3):
    return pl.pallas_call(
        _s3_body,
        grid=(GRID_E,),
        in_specs=[_EB, _EB, _CONST2, _CONST1, _CONST1, _WBD],
        out_specs=[_EB, _EB, _CONST2],
        out_shape=[_E16, _E16, _ST],
        scratch_shapes=[pltpu.VMEM((2, 128), jnp.float32)],
        compiler_params=_TC_PARAMS,
    )(a2, h1, st2, g2, be2, w3)


def _s4_body(a3, h2, st3, g3, be3, u1, u2, efo, a4, a5, st4, st5, acc):
    i = pl.program_id(0)
    m, s, b = _finalize(st3, g3, be3)
    ef = _lrelu((a3[...] - m) * s + b) + h2[...]
    efo[...] = ef
    x4 = ef @ u1[...]
    a4[...] = x4
    _acc_stats(acc, x4, 0, i == 0, i == GRID_E - 1, st4)
    x5 = ef @ u2[...]
    a5[...] = x5
    _acc_stats(acc, x5, 1, i == 0, i == GRID_E - 1, st5)


def _s4(a3, h2, st3, g3, be3, u1, u2):
    return pl.pallas_call(
        _s4_body,
        grid=(GRID_E,),
        in_specs=[_EB, _EB, _CONST2, _CONST1, _CONST1, _WBD, _WBD],
        out_specs=[_EB, _EB, _EB, _CONST2, _CONST2],
        out_shape=[_E16, _E16, _E16, _ST, _ST],
        scratch_shapes=[pltpu.VMEM((4, 128), jnp.float32)],
        compiler_params=_TC_PARAMS,
    )(a3, h2, st3, g3, be3, u1, u2)


def _s5_body(a4, a5, ef, st4, g4, be4, st5, g5, be5, wos, bos, msg1, h5o):
    m4, s4, b4 = _finalize(st4, g4, be4)
    efv = ef[...]
    h4 = _lrelu((a4[...] - m4) * s4 + b4) + efv
    msg1[...] = h4 @ wos[...] + bos[...]
    m5, s5, b5 = _finalize(st5, g5, be5)
    h5o[...] = _lrelu((a5[...] - m5) * s5 + b5) + efv


def _s5(a4, a5, ef, st4, g4, be4, st5, g5, be5, wos, bos):
    return pl.pallas_call(
        _s5_body,
        grid=(GRID_E,),
        in_specs=[_EB, _EB, _EB, _CONST2, _CONST1, _CONST1,
                  _CONST2, _CONST1, _CONST1, _WBD, _CONST1],
        out_specs=[_EB, _EB],
        out_shape=[_E16, _E16],
        compiler_params=_TC_PARAMS,
    )(a4, a5, ef, st4, g4, be4, st5, g5, be5, wos, bos)


def _nstats(t):
    rows = lax.broadcasted_iota(jnp.int32, (NP, 128), 0)
    tm = jnp.where(rows < N // 8, t, 0.0)
    m = _fold(jnp.sum(tm, axis=0, keepdims=True)) * (1.0 / N)
    q = _fold(jnp.sum(tm * tm, axis=0, keepdims=True)) * (1.0 / N)
    return m, q


def _na_body(p0, p1, g, be, rw2, out1o, root2o):
    aggr = p0[...] + p1[...]
    m, q = _nstats(aggr)
    inv = lax.rsqrt(q - m * m + EPS)
    out1 = _lrelu((aggr - m) * (inv * g[...]) + be[...]) + 1.0
    out1o[...] = out1
    root2o[...] = out1 @ rw2[...]


def _na(p0, p1, g, be, rw2):
    nb = pl.BlockSpec((NP, 128), _const)
    return pl.pallas_call(
        _na_body,
        grid=(1,),
        in_specs=[nb, nb, _CONST1, _CONST1, _WBD],
        out_specs=[nb, nb],
        out_shape=[jax.ShapeDtypeStruct((NP, 128), jnp.float32),
                   jax.ShapeDtypeStruct((NP, 128), jnp.float32)],
    )(p0, p1, g, be, rw2)


def _s10_body(h5, xs2, wo2, bo2, msg2):
    h5v = h5[...]
    xsv = xs2[...]
    for j in range(8):
        h5j = h5v[:, D * j:D * j + D]
        xsj = xsv[:, D * j:D * j + D]
        wblk = h5j @ wo2[...] + bo2[...]
        acc = xsj[:, 0:1] * wblk[:, 0:D]
        for i in range(1, D):
            acc = acc + xsj[:, i:i + 1] * wblk[:, D * i:D * i + D]
        msg2[:, D * j:D * j + D] = acc


def _s10(h5, xs2, wo2, bo2):
    return pl.pallas_call(
        _s10_body,
        grid=(GRID_E,),
        in_specs=[_EB, _EB, pl.BlockSpec((D, 16 * D), _const),
                  pl.BlockSpec((1, 16 * D), _const)],
        out_specs=[_EB],
        out_shape=[_E16],
        compiler_params=_TC_PARAMS,
    )(h5, xs2, wo2, bo2)


def _nb_body(p0, p1, root2, out1, g, be, out2o):
    t = p0[...] + p1[...] + root2[...]
    m, q = _nstats(t)
    inv = lax.rsqrt(q - m * m + EPS)
    out2o[...] = _lrelu((t - m) * (inv * g[...]) + be[...]) + out1[...]


def _nb(p0, p1, root2, out1, g, be):
    nb = pl.BlockSpec((NP, 128), _const)
    return pl.pallas_call(
        _nb_body,
        grid=(1,),
        in_specs=[nb, nb, nb, nb, _CONST1, _CONST1],
        out_specs=nb,
        out_shape=jax.ShapeDtypeStruct((NP, 128), jnp.float32),
    )(p0, p1, root2, out1, g, be)


_SC_MESH = plsc.VectorSubcoreMesh(core_axis_name="core", subcore_axis_name="subcore")
_SC_PARAMS = pltpu.CompilerParams(use_tc_tiling_on_sc=False)


def _gather2(table, src3, dst3):
    @functools.partial(
        pl.kernel,
        out_type=[jax.ShapeDtypeStruct((E_PAD, D), jnp.float32),
                  jax.ShapeDtypeStruct((E_PAD, D), jnp.float32)],
        mesh=_SC_MESH,
        scratch_types=[pltpu.VMEM((WPW, 1, WIN), jnp.int32),
                       pltpu.VMEM((WPW, 1, WIN), jnp.int32),
                       pltpu.VMEM((CH * WIN, D), jnp.float32),
                       pltpu.VMEM((CH * WIN, D), jnp.float32),
                       pltpu.SemaphoreType.DMA,
                       pltpu.SemaphoreType.DMA],
        compiler_params=_SC_PARAMS,
    )
    def k(tab_hbm, src_hbm, dst_hbm, xs_hbm, xd_hbm, si_v, di_v, ga_v, gb_v,
          sem_g, sem_o):
        c = lax.axis_index("core")
        s = lax.axis_index("subcore")
        w = c * 16 + s
        base = pl.multiple_of(w * WPW, 8)
        pltpu.sync_copy(src_hbm.at[pl.ds(base, WPW)], si_v)
        pltpu.sync_copy(dst_hbm.at[pl.ds(base, WPW)], di_v)
        outs = []
        for ch in range(WPW // CH):
            if outs:
                for o in outs:
                    o.wait()
                outs = []
            descs = []
            for kk in range(CH):
                j = ch * CH + kk
                descs.append(pltpu.async_copy(
                    tab_hbm.at[si_v.at[j, 0]],
                    ga_v.at[pl.ds(kk * WIN, WIN)], sem_g))
                descs.append(pltpu.async_copy(
                    tab_hbm.at[di_v.at[j, 0]],
                    gb_v.at[pl.ds(kk * WIN, WIN)], sem_g))
            for dsc in descs:
                dsc.wait()
            row = pl.multiple_of((base + ch * CH) * WIN, 8)
            outs = [pltpu.async_copy(ga_v, xs_hbm.at[pl.ds(row, CH * WIN)], sem_o),
                    pltpu.async_copy(gb_v, xd_hbm.at[pl.ds(row, CH * WIN)], sem_o)]
        for o in outs:
            o.wait()

    return k(table, src3, dst3)


def _gather1(table, idx3):
    @functools.partial(
        pl.kernel,
        out_type=jax.ShapeDtypeStruct((E_PAD, D), jnp.float32),
        mesh=_SC_MESH,
        scratch_types=[pltpu.VMEM((WPW, 1, WIN), jnp.int32),
                       pltpu.VMEM((WPW * WIN, D), jnp.float32),
                       pltpu.SemaphoreType.DMA],
        compiler_params=_SC_PARAMS,
    )
    def k(tab_hbm, idx_hbm, out_hbm, i_v, g_v, sem_g):
        c = lax.axis_index("core")
        s = lax.axis_index("subcore")
        w = c * 16 + s
        base = pl.multiple_of(w * WPW, 8)
        pltpu.sync_copy(idx_hbm.at[pl.ds(base, WPW)], i_v)
        descs = []
        for j in range(WPW):
            descs.append(pltpu.async_copy(
                tab_hbm.at[i_v.at[j, 0]],
                g_v.at[pl.ds(j * WIN, WIN)], sem_g))
        for dsc in descs:
            dsc.wait()
        row = pl.multiple_of(base * WIN, 8)
        pltpu.sync_copy(g_v, out_hbm.at[pl.ds(row, WPW * WIN)])

    return k(table, idx3)


def _scatter_add(msg, dst3, zeros):
    @functools.partial(
        pl.kernel,
        out_type=[jax.ShapeDtypeStruct((N_PAD, D), jnp.float32),
                  jax.ShapeDtypeStruct((N_PAD, D), jnp.float32)],
        mesh=_SC_MESH,
        scratch_types=[pltpu.VMEM((WPW, 1, WIN), jnp.int32),
                       pltpu.VMEM((CH * WIN, D), jnp.float32),
                       pltpu.VMEM((CH * WIN, D), jnp.float32),
                       pltpu.VMEM((NR, D), jnp.float32),
                       pltpu.SemaphoreType.DMA,
                       pltpu.SemaphoreType.DMA,
                       pltpu.VMEM_SHARED((N_PAD, D), jnp.float32)],
        compiler_params=_SC_PARAMS,
    )
    def k(msg_hbm, dst_hbm, z_hbm, p0_hbm, p1_hbm, idx_v, ma_v, mb_v, bounce_v,
          sem_m, sem_a, acc_sh):
        c = lax.axis_index("core")
        s = lax.axis_index("subcore")
        w = c * 16 + s
        base = pl.multiple_of(w * WPW, 8)
        pltpu.sync_copy(dst_hbm.at[pl.ds(base, WPW)], idx_v)
        row0 = pl.multiple_of(s * NR, 8)
        pltpu.sync_copy(z_hbm.at[pl.ds(row0, NR)], acc_sh.at[pl.ds(row0, NR)])
        plsc.subcore_barrier()

        bufs = (ma_v, mb_v)
        loads = []
        for ch in range(WPW // CH):
            for kk in range(CH):
                j = ch * CH + kk
                win = jnp.minimum(base + j, NWIN - 1)
                e0 = pl.multiple_of(win * WIN, WIN)
                loads.append(pltpu.async_copy(
                    msg_hbm.at[pl.ds(e0, WIN)],
                    bufs[ch].at[pl.ds(kk * WIN, WIN)], sem_m))
        adds = []
        for ch in range(WPW // CH):
            for kk in range(CH):
                j = ch * CH + kk
                loads[j].wait()
                adds.append(pltpu.async_copy(
                    bufs[ch].at[pl.ds(kk * WIN, WIN)],
                    acc_sh.at[idx_v.at[j, 0]], sem_a, add=True))
        for a in adds:
            a.wait()

        plsc.subcore_barrier()
        pltpu.sync_copy(acc_sh.at[pl.ds(row0, NR)], bounce_v)

        @pl.when(c == 0)
        def _():
            pltpu.sync_copy(bounce_v, p0_hbm.at[pl.ds(row0, NR)])

        @pl.when(c == 1)
        def _():
            pltpu.sync_copy(bounce_v, p1_hbm.at[pl.ds(row0, NR)])

    return k(msg, dst3, zeros)


def _bd8(w):
    return jnp.kron(jnp.eye(8, dtype=jnp.float32), w)


def _tile8(v):
    return jnp.tile(v.reshape(1, D), (1, 8))


def kernel(node_feat, edge_index, edge_attr, batch_index, shared_net, layers):
    del batch_index
    (w1, _, g1, be1), (w2, _, g2, be2), (w3, _, g3, be3) = shared_net
    w1a, w1b, w1c = w1[:4], w1[4:4 + D], w1[4 + D:4 + 2 * D]
    l1, l2 = layers
    (u1, _, ug1, ube1), = l1["en_hidden"]
    (u2, _, ug2, ube2), = l2["en_hidden"]
    wo1, bo1 = l1["en_out"]
    wos = wo1.reshape(D, D, D).sum(axis=1)
    bos = bo1.reshape(D, D).sum(axis=0)
    wo2, bo2 = l2["en_out"]
    bo2r = bo2.reshape(1, D * D)

    g1, be1, g2, be2, g3, be3 = map(_tile8, (g1, be1, g2, be2, g3, be3))
    ug1, ube1, ug2, ube2 = map(_tile8, (ug1, ube1, ug2, ube2))
    n1g, n1b = _tile8(l1["bn_g"]), _tile8(l1["bn_b"])
    n2g, n2b = _tile8(l2["bn_g"]), _tile8(l2["bn_b"])

    src3 = jnp.pad(edge_index[0], (0, E_PAD - E)).reshape(NWIN_PAD, 1, WIN)
    dst3g = jnp.pad(edge_index[1], (0, E_PAD - E)).reshape(NWIN_PAD, 1, WIN)
    dst3s = jnp.pad(edge_index[1], (0, E_PAD - E),
                    constant_values=N_PAD - 1).reshape(NWIN_PAD, 1, WIN)
    zeros = jnp.zeros((N_PAD, D), jnp.float32)
    ea_p = edge_attr.reshape(EP, 32)

    pk = lambda x: x.reshape(E_PAD // 8, 128)
    pkn = lambda x: x.reshape(NP, 128)

    xs, xd = _gather2(node_feat, src3, dst3g)
    a1, st1 = _s1(ea_p, pk(xs), pk(xd), _bd8(w1a), _bd8(w1b), _bd8(w1c))
    a2, h1, st2 = _s2(a1, st1, g1, be1, _bd8(w2))
    a3, h2, st3 = _s3(a2, h1, st2, g2, be2, _bd8(w3))
    ef, a4, a5, st4, st5 = _s4(a3, h2, st3, g3, be3, _bd8(u1), _bd8(u2))
    msg1, h5 = _s5(a4, a5, ef, st4, ug1, ube1, st5, ug2, ube2,
                   _bd8(wos), _tile8(bos))
    p0, p1 = _scatter_add(msg1.reshape(E, D), dst3s, zeros)
    out1, root2 = _na(pkn(p0), pkn(p1), n1g, n1b, _bd8(l2["root_W"]))
    xs2 = _gather1(out1.reshape(N_PAD, D), src3)
    msg2, = _s10(h5, pk(xs2), wo2, bo2r)
    q0, q1 = _scatter_add(msg2.reshape(E, D), dst3s, zeros)
    out2 = _nb(pkn(q0), pkn(q1), root2, out1, n2g, n2b)
    return out2.reshape(N_PAD, D)[:N]
